# baseline (speedup 1.0000x reference)
import jax
import jax.numpy as jnp
from jax.experimental import pallas as pl
from jax.experimental.pallas import tpu as pltpu


def _matmul_body(x_ref, w_ref, s_ref, b_ref, o_ref):
    bn, kdim = w_ref.shape
    group = 128
    n_groups = kdim // group
    parts = []
    for g in range(n_groups):
        wg = w_ref[:, g * group:(g + 1) * group].astype(jnp.float32)
        parts.append((wg * s_ref[:, g:g + 1]).astype(jnp.bfloat16))
    w_bf = jnp.concatenate(parts, axis=1)
    o_ref[...] = jax.lax.dot_general(
        x_ref[...], w_bf,
        dimension_numbers=(((1,), (1,)), ((), ())),
        preferred_element_type=jnp.float32,
    ) + b_ref[...]


def _quant_matmul(x_bf, w2, s2, b_row, *, bm, bn):
    m, kdim = x_bf.shape
    n, n_groups = s2.shape
    grid = (pl.cdiv(m, bm), pl.cdiv(n, bn))
    return pl.pallas_call(
        _matmul_body,
        grid=grid,
        in_specs=[
            pl.BlockSpec((bm, kdim), lambda mi, ni: (mi, 0)),
            pl.BlockSpec((bn, kdim), lambda mi, ni: (ni, 0)),
            pl.BlockSpec((bn, n_groups), lambda mi, ni: (ni, 0)),
            pl.BlockSpec((1, bn), lambda mi, ni: (0, ni)),
        ],
        out_specs=pl.BlockSpec((bm, bn), lambda mi, ni: (mi, ni)),
        out_shape=jax.ShapeDtypeStruct((m, n), jnp.float32),
        compiler_params=pltpu.CompilerParams(
            dimension_semantics=("parallel", "parallel"),
        ),
    )(x_bf, w2, s2, b_row)


def kernel(x, w_q, scales, bias):
    out_f, n_groups, group = w_q.shape
    m, in_f = x.shape
    w2 = w_q.reshape(out_f, in_f)
    s2 = scales.reshape(out_f, n_groups)
    b_row = bias.reshape(1, out_f)
    x_bf = x.astype(jnp.bfloat16)
    y = _quant_matmul(x_bf, w2, s2, b_row, bm=2048, bn=512)
    return y.astype(x.dtype)
